# baseline (speedup 1.0000x reference)
import functools

import jax
import jax.numpy as jnp
from jax import lax
from jax.experimental import pallas as pl
from jax.experimental.pallas import tpu as pltpu
from jax.experimental.pallas import tpu_sc as plsc

N_NODES = 10000
E = 320000
D = 128
N_CELLS = 100
N_DRUGS = 2000
FP_DIM = 1024
B = 64

NC = 2
NS = 16
NW = NC * NS
EPW = E // NW
CH = 80
NFULL = EPW // CH
RCH = 80
NRCH = N_NODES // RCH
ZCH = 40
NZCH = N_NODES // ZCH


def _pre_body(node_emb, W_g, a2, ctl, W_ctl, h_out, hsd_out, ctlp_out):
    h = jnp.dot(node_emb[...], W_g[...], preferred_element_type=jnp.float32)
    h_out[...] = h
    hsd_out[...] = jnp.dot(h, a2[...], preferred_element_type=jnp.float32)
    ctlp_out[...] = jnp.dot(ctl[...], W_ctl[...],
                            preferred_element_type=jnp.float32)


def _pre_call(node_emb, W_g, a2, ctl, W_ctl):
    return pl.pallas_call(
        _pre_body,
        out_shape=[
            jax.ShapeDtypeStruct((N_NODES, D), jnp.float32),
            jax.ShapeDtypeStruct((N_NODES, 2), jnp.float32),
            jax.ShapeDtypeStruct((B, D), jnp.float32),
        ],
    )(node_emb, W_g, a2, ctl, W_ctl)


def _edge_body(src_h, dst_h, ew_h, hs_h, hd_h, h_h, agg_out, den_out,
               hs_v, hd_v, srcbA, srcbB, ewbA, ewbB, rowsA, rowsB,
               dstb0, dstb1, dstb2, pvec0, pvec1, pvec2,
               semA, semB, sem2, ssem, agg_sh, den_sh):
    c = lax.axis_index("c")
    s = lax.axis_index("s")
    wid = c * NS + s

    srcbs = (srcbA, srcbB)
    ewbs = (ewbA, ewbB)
    rowss = (rowsA, rowsB)
    gsems = (semA, semB)
    dstbs = (dstb0, dstb1, dstb2)
    pvecs = (pvec0, pvec1, pvec2)

    pltpu.sync_copy(hs_h, hs_v)
    pltpu.sync_copy(hd_h, hd_v)

    zero16 = jnp.zeros((16,), jnp.float32)

    def _zr(i, _):
        for j in range(8):
            rowsA[i, pl.ds(16 * j, 16)] = zero16
            rowsB[i, pl.ds(16 * j, 16)] = zero16
        return 0

    lax.fori_loop(0, CH, _zr, 0)
    for j in range(CH // 16):
        pvec0[pl.ds(16 * j, 16)] = zero16
        pvec1[pl.ds(16 * j, 16)] = zero16
        pvec2[pl.ds(16 * j, 16)] = zero16

    for m in range(16):
        k = s + 16 * m

        @pl.when(k < NZCH)
        def _():
            pltpu.sync_copy(rowsA.at[pl.ds(0, ZCH)],
                            agg_sh.at[pl.ds(ZCH * k, ZCH)])
            pltpu.sync_copy(pvec0.at[pl.ds(0, ZCH)],
                            den_sh.at[pl.ds(ZCH * k, ZCH)])

    plsc.subcore_barrier()

    ebase = wid * EPW

    def _ecomp(srcb, dstb, ewb, pvec, n16):
        @plsc.parallel_loop(0, n16)
        def _eb(j):
            sl = pl.ds(16 * j, 16)
            s16 = srcb[sl]
            d16 = dstb[sl]
            hs16 = plsc.load_gather(hs_v, [s16])
            hd16 = plsc.load_gather(hd_v, [d16])
            x = hs16 + hd16
            e = jnp.maximum(x, 0.2 * x)
            p = jnp.exp(e) * ewb[sl]
            pvec[sl] = p

    def _scale(rows, pvec, n16):
        @plsc.parallel_loop(0, n16)
        def _sgroup(g):
            pv = pvec[pl.ds(16 * g, 16)]
            for l in range(16):
                pi = pv[l]
                r = 16 * g + l
                for j in range(8):
                    rsl = pl.ds(16 * j, 16)
                    rows[r, rsl] = rows[r, rsl] * pi

    def _phase(p, g):
        ci = 6 * g + p
        la, tri = p % 2, p % 3
        pla, ptri = (p - 1) % 2, (p - 1) % 3
        nla, ntri = (p + 1) % 2, (p + 1) % 3
        srcb, ewb, rows, gsem = srcbs[la], ewbs[la], rowss[la], gsems[la]
        dstb, pvec = dstbs[tri], pvecs[tri]
        pltpu.make_async_copy(rowss[pla], agg_sh.at[dstbs[ptri]], ssem).wait()
        pltpu.make_async_copy(pvecs[ptri], den_sh.at[dstbs[ptri]], ssem).wait()
        ni = ci + 1
        noff = ebase + jnp.where(ni < NFULL, ni, 0) * CH
        cpa = pltpu.async_copy(src_h.at[pl.ds(noff, CH)], srcbs[nla], sem2)
        cpb = pltpu.async_copy(dst_h.at[pl.ds(noff, CH)], dstbs[ntri], sem2)
        cpc = pltpu.async_copy(ew_h.at[pl.ds(noff, CH)], ewbs[nla], sem2)
        _ecomp(srcb, dstb, ewb, pvec, CH // 16)
        cpa.wait()
        cpb.wait()
        cpc.wait()
        pltpu.async_copy(h_h.at[srcbs[nla]], rowss[nla], gsems[nla])
        pltpu.make_async_copy(h_h.at[srcb], rows, gsem).wait()
        _scale(rows, pvec, CH // 16)
        pltpu.async_copy(rows, agg_sh.at[dstb], ssem, add=True)
        pltpu.async_copy(pvec, den_sh.at[dstb], ssem, add=True)

    pltpu.sync_copy(src_h.at[pl.ds(ebase, CH)], srcbA)
    pltpu.sync_copy(dst_h.at[pl.ds(ebase, CH)], dstb0)
    pltpu.sync_copy(ew_h.at[pl.ds(ebase, CH)], ewbA)
    pltpu.sync_copy(dst_h.at[pl.ds(ebase, CH)], dstb2)
    pltpu.async_copy(rowsB, agg_sh.at[dstb2], ssem, add=True)
    pltpu.async_copy(pvec2, den_sh.at[dstb2], ssem, add=True)
    pltpu.async_copy(h_h.at[srcbA], rowsA, semA)

    def _six(g, _):
        for p in range(6):
            _phase(p, g)
        return 0

    lax.fori_loop(0, NFULL // 6, _six, 0)
    for p in range(NFULL % 6):
        _phase(p, NFULL // 6)

    pltpu.make_async_copy(rowsA, agg_sh.at[dstb1], ssem).wait()
    pltpu.make_async_copy(pvec1, den_sh.at[dstb1], ssem).wait()
    pltpu.make_async_copy(h_h.at[srcbB], rowsB, semB).wait()

    plsc.subcore_barrier()

    for m in range(8):
        k = s + 16 * m

        @pl.when(k < NRCH)
        def _():
            pltpu.sync_copy(agg_sh.at[pl.ds(RCH * k, RCH)],
                            agg_out.at[c, pl.ds(RCH * k, RCH)])
            pltpu.sync_copy(den_sh.at[pl.ds(RCH * k, RCH)],
                            den_out.at[pl.ds(c * N_NODES + RCH * k, RCH)])


@functools.partial(
    pl.kernel,
    out_type=[
        jax.ShapeDtypeStruct((NC, N_NODES, D), jnp.float32),
        jax.ShapeDtypeStruct((NC * N_NODES,), jnp.float32),
    ],
    mesh=plsc.VectorSubcoreMesh(core_axis_name="c", subcore_axis_name="s",
                                num_cores=NC, num_subcores=NS),
    compiler_params=pltpu.CompilerParams(needs_layout_passes=False,
                                         use_tc_tiling_on_sc=False),
    scratch_types=[
        pltpu.VMEM((N_NODES,), jnp.float32),
        pltpu.VMEM((N_NODES,), jnp.float32),
        pltpu.VMEM((CH,), jnp.int32),
        pltpu.VMEM((CH,), jnp.int32),
        pltpu.VMEM((CH,), jnp.float32),
        pltpu.VMEM((CH,), jnp.float32),
        pltpu.VMEM((CH, D), jnp.float32),
        pltpu.VMEM((CH, D), jnp.float32),
        pltpu.VMEM((CH,), jnp.int32),
        pltpu.VMEM((CH,), jnp.int32),
        pltpu.VMEM((CH,), jnp.int32),
        pltpu.VMEM((CH,), jnp.float32),
        pltpu.VMEM((CH,), jnp.float32),
        pltpu.VMEM((CH,), jnp.float32),
        pltpu.SemaphoreType.DMA,
        pltpu.SemaphoreType.DMA,
        pltpu.SemaphoreType.DMA,
        pltpu.SemaphoreType.DMA,
        pltpu.VMEM_SHARED((N_NODES, D), jnp.float32),
        pltpu.VMEM_SHARED((N_NODES,), jnp.float32),
    ],
)
def _edge_call(src_h, dst_h, ew_h, hs_h, hd_h, h_h, agg_out, den_out,
               *scratch):
    _edge_body(src_h, dst_h, ew_h, hs_h, hd_h, h_h, agg_out, den_out,
               *scratch)


def _post_body(agg_ref, den_ref, dt_ref, ctlp_ref, dfp_ref, cid_ref,
               fpt_ref, ct_ref, Wfp_ref, Wout_ref, out_ref):
    aggs = agg_ref[0] + agg_ref[1]
    dens = den_ref[0] + den_ref[1] + 1e-9
    g = aggs / dens
    gene = jnp.where(g > 0, g, jnp.exp(g) - 1.0)
    dt = jnp.dot(dt_ref[...], gene, preferred_element_type=jnp.float32)
    oh_fp = (dfp_ref[...] == lax.broadcasted_iota(
        jnp.int32, (B, N_DRUGS), 1)).astype(jnp.float32)
    fpf = jnp.dot(oh_fp, fpt_ref[...], preferred_element_type=jnp.float32)
    fpe = jnp.dot(fpf, Wfp_ref[...], preferred_element_type=jnp.float32)
    oh_c = (cid_ref[...] == lax.broadcasted_iota(
        jnp.int32, (B, N_CELLS), 1)).astype(jnp.float32)
    cellr = jnp.dot(oh_c, ct_ref[...], preferred_element_type=jnp.float32)
    z = jnp.maximum(dt + ctlp_ref[...] + cellr + fpe, 0.0)
    out_ref[...] = jnp.dot(z, Wout_ref[...],
                           preferred_element_type=jnp.float32)


def _post_call(aggU, den3, drug_targets, ctlp, dfp2, cid2, fp_table,
               cell_table, W_fp, W_out):
    return pl.pallas_call(
        _post_body,
        out_shape=jax.ShapeDtypeStruct((B, N_NODES), jnp.float32),
    )(aggU, den3, drug_targets, ctlp, dfp2, cid2, fp_table, cell_table,
      W_fp, W_out)


def kernel(ctl, drug_targets, cell_idx, drug_fp, edge_index, edge_weight,
           fp_table, node_emb, W_g, a_s, a_d, W_ctl, W_fp, cell_table, W_out):
    src = edge_index[0]
    dst = edge_index[1]
    a2 = jnp.stack([a_s, a_d], axis=1)
    h, hsd, ctlp = _pre_call(node_emb, W_g, a2, ctl, W_ctl)
    hs = hsd[:, 0]
    hd = hsd[:, 1]
    aggU, den_flat = _edge_call(src, dst, edge_weight, hs, hd, h)
    den3 = den_flat.reshape(NC, N_NODES, 1)
    dfp2 = drug_fp.astype(jnp.int32).reshape(B, 1)
    cid2 = cell_idx.astype(jnp.int32).reshape(B, 1)
    return _post_call(aggU, den3, drug_targets, ctlp, dfp2, cid2, fp_table,
                      cell_table, W_fp, W_out)


# speedup vs baseline: 1.1831x; 1.1831x over previous
import functools

import jax
import jax.numpy as jnp
from jax import lax
from jax.experimental import pallas as pl
from jax.experimental.pallas import tpu as pltpu
from jax.experimental.pallas import tpu_sc as plsc

N_NODES = 10000
E = 320000
D = 128
N_CELLS = 100
N_DRUGS = 2000
FP_DIM = 1024
B = 64

NC = 2
NS = 16
NW = NC * NS
EPW = E // NW
CH = 80
NFULL = EPW // CH
RCH = 80
NRCH = N_NODES // RCH
ZCH = 40
NZCH = N_NODES // ZCH


def _pre_body(node_emb, W_g, a2, ctl, W_ctl, h_out, hsd_out, ctlp_out):
    h = jnp.dot(node_emb[...], W_g[...], preferred_element_type=jnp.float32)
    h_out[...] = h
    hsd_out[...] = jnp.dot(h, a2[...], preferred_element_type=jnp.float32)
    ctlp_out[...] = jnp.dot(ctl[...], W_ctl[...],
                            preferred_element_type=jnp.float32)


def _pre_call(node_emb, W_g, a2, ctl, W_ctl):
    return pl.pallas_call(
        _pre_body,
        out_shape=[
            jax.ShapeDtypeStruct((N_NODES, D), jnp.float32),
            jax.ShapeDtypeStruct((N_NODES, 2), jnp.float32),
            jax.ShapeDtypeStruct((B, D), jnp.float32),
        ],
    )(node_emb, W_g, a2, ctl, W_ctl)


def _edge_body(src_h, dst_h, ew_h, hs_h, hd_h, h_h, agg_out, den_out,
               hs_v, hd_v, srcbA, srcbB, ewbA, ewbB, rowsA, rowsB,
               dstb0, dstb1, dstb2, pvec0, pvec1, pvec2,
               semA, semB, sem2, ssem, agg_sh, den_sh):
    c = lax.axis_index("c")
    s = lax.axis_index("s")
    wid = c * NS + s

    srcbs = (srcbA, srcbB)
    ewbs = (ewbA, ewbB)
    rowss = (rowsA, rowsB)
    gsems = (semA, semB)
    dstbs = (dstb0, dstb1, dstb2)
    pvecs = (pvec0, pvec1, pvec2)

    pltpu.sync_copy(hs_h, hs_v)
    pltpu.sync_copy(hd_h, hd_v)

    zero16 = jnp.zeros((16,), jnp.float32)

    def _zr(i, _):
        for j in range(8):
            rowsA[i, pl.ds(16 * j, 16)] = zero16
            rowsB[i, pl.ds(16 * j, 16)] = zero16
        return 0

    lax.fori_loop(0, CH, _zr, 0)
    for j in range(CH // 16):
        pvec0[pl.ds(16 * j, 16)] = zero16
        pvec1[pl.ds(16 * j, 16)] = zero16
        pvec2[pl.ds(16 * j, 16)] = zero16

    for m in range(16):
        k = s + 16 * m

        @pl.when(k < NZCH)
        def _():
            pltpu.sync_copy(rowsA.at[pl.ds(0, ZCH)],
                            agg_sh.at[pl.ds(ZCH * k, ZCH)])
            pltpu.sync_copy(pvec0.at[pl.ds(0, ZCH)],
                            den_sh.at[pl.ds(ZCH * k, ZCH)])

    plsc.subcore_barrier()

    ebase = wid * EPW

    def _ecomp(srcb, dstb, ewb, pvec, n16):
        @plsc.parallel_loop(0, n16)
        def _eb(j):
            sl = pl.ds(16 * j, 16)
            s16 = srcb[sl]
            d16 = dstb[sl]
            hs16 = plsc.load_gather(hs_v, [s16])
            hd16 = plsc.load_gather(hd_v, [d16])
            x = hs16 + hd16
            e = jnp.maximum(x, 0.2 * x)
            p = jnp.exp(e) * ewb[sl]
            pvec[sl] = p

    def _scale(rows, pvec, n16):
        @plsc.parallel_loop(0, n16)
        def _sgroup(g):
            pv = pvec[pl.ds(16 * g, 16)]
            for l in range(16):
                pi = pv[l]
                r = 16 * g + l
                for j in range(8):
                    rsl = pl.ds(16 * j, 16)
                    rows[r, rsl] = rows[r, rsl] * pi

    def _scale_scatter(rows, pvec, dstb):
        _scale(rows, pvec, CH // 16)
        pltpu.sync_copy(rows, agg_sh.at[dstb], add=True)
        pltpu.sync_copy(pvec, den_sh.at[dstb], add=True)

    bufA = (srcbA, dstb0, ewbA, pvec0, rowsA, semA)
    bufB = (srcbB, dstb1, ewbB, pvec1, rowsB, semB)

    def _phase(ci, ni, cur, nxt):
        srcb, dstb, ewb, pvec, rows, gsem = cur
        if ni is not None:
            srcb_n, dstb_n, ewb_n, _, rows_n, gsem_n = nxt
            noff = ebase + ni * CH
            cpa = pltpu.async_copy(src_h.at[pl.ds(noff, CH)], srcb_n, sem2)
            cpb = pltpu.async_copy(dst_h.at[pl.ds(noff, CH)], dstb_n, sem2)
            cpc = pltpu.async_copy(ew_h.at[pl.ds(noff, CH)], ewb_n, sem2)
        _ecomp(srcb, dstb, ewb, pvec, CH // 16)
        if ni is not None:
            cpa.wait()
            cpb.wait()
            cpc.wait()
            pltpu.async_copy(h_h.at[srcb_n], rows_n, gsem_n)
        pltpu.make_async_copy(h_h.at[srcb], rows, gsem).wait()
        _scale_scatter(rows, pvec, dstb)

    pltpu.sync_copy(src_h.at[pl.ds(ebase, CH)], srcbA)
    pltpu.sync_copy(dst_h.at[pl.ds(ebase, CH)], dstb0)
    pltpu.sync_copy(ew_h.at[pl.ds(ebase, CH)], ewbA)
    pltpu.async_copy(h_h.at[srcbA], rowsA, semA)

    def _pair(g, _):
        ci = 2 * g
        _phase(ci, ci + 1, bufA, bufB)
        _phase(ci + 1, ci + 2, bufB, bufA)
        return 0

    lax.fori_loop(0, (NFULL - 1) // 2, _pair, 0)
    _phase(NFULL - 1, None, bufA, None)

    plsc.subcore_barrier()

    for m in range(8):
        k = s + 16 * m

        @pl.when(k < NRCH)
        def _():
            pltpu.sync_copy(agg_sh.at[pl.ds(RCH * k, RCH)],
                            agg_out.at[c, pl.ds(RCH * k, RCH)])
            pltpu.sync_copy(den_sh.at[pl.ds(RCH * k, RCH)],
                            den_out.at[pl.ds(c * N_NODES + RCH * k, RCH)])


@functools.partial(
    pl.kernel,
    out_type=[
        jax.ShapeDtypeStruct((NC, N_NODES, D), jnp.float32),
        jax.ShapeDtypeStruct((NC * N_NODES,), jnp.float32),
    ],
    mesh=plsc.VectorSubcoreMesh(core_axis_name="c", subcore_axis_name="s",
                                num_cores=NC, num_subcores=NS),
    compiler_params=pltpu.CompilerParams(needs_layout_passes=False,
                                         use_tc_tiling_on_sc=False),
    scratch_types=[
        pltpu.VMEM((N_NODES,), jnp.float32),
        pltpu.VMEM((N_NODES,), jnp.float32),
        pltpu.VMEM((CH,), jnp.int32),
        pltpu.VMEM((CH,), jnp.int32),
        pltpu.VMEM((CH,), jnp.float32),
        pltpu.VMEM((CH,), jnp.float32),
        pltpu.VMEM((CH, D), jnp.float32),
        pltpu.VMEM((CH, D), jnp.float32),
        pltpu.VMEM((CH,), jnp.int32),
        pltpu.VMEM((CH,), jnp.int32),
        pltpu.VMEM((CH,), jnp.int32),
        pltpu.VMEM((CH,), jnp.float32),
        pltpu.VMEM((CH,), jnp.float32),
        pltpu.VMEM((CH,), jnp.float32),
        pltpu.SemaphoreType.DMA,
        pltpu.SemaphoreType.DMA,
        pltpu.SemaphoreType.DMA,
        pltpu.SemaphoreType.DMA,
        pltpu.VMEM_SHARED((N_NODES, D), jnp.float32),
        pltpu.VMEM_SHARED((N_NODES,), jnp.float32),
    ],
)
def _edge_call(src_h, dst_h, ew_h, hs_h, hd_h, h_h, agg_out, den_out,
               *scratch):
    _edge_body(src_h, dst_h, ew_h, hs_h, hd_h, h_h, agg_out, den_out,
               *scratch)


def _post_body(agg_ref, den_ref, dt_ref, ctlp_ref, dfp_ref, cid_ref,
               fpt_ref, ct_ref, Wfp_ref, Wout_ref, out_ref):
    aggs = agg_ref[0] + agg_ref[1]
    dens = den_ref[0] + den_ref[1] + 1e-9
    g = aggs / dens
    gene = jnp.where(g > 0, g, jnp.exp(g) - 1.0)
    dt = jnp.dot(dt_ref[...], gene, preferred_element_type=jnp.float32)
    oh_fp = (dfp_ref[...] == lax.broadcasted_iota(
        jnp.int32, (B, N_DRUGS), 1)).astype(jnp.float32)
    fpf = jnp.dot(oh_fp, fpt_ref[...], preferred_element_type=jnp.float32)
    fpe = jnp.dot(fpf, Wfp_ref[...], preferred_element_type=jnp.float32)
    oh_c = (cid_ref[...] == lax.broadcasted_iota(
        jnp.int32, (B, N_CELLS), 1)).astype(jnp.float32)
    cellr = jnp.dot(oh_c, ct_ref[...], preferred_element_type=jnp.float32)
    z = jnp.maximum(dt + ctlp_ref[...] + cellr + fpe, 0.0)
    out_ref[...] = jnp.dot(z, Wout_ref[...],
                           preferred_element_type=jnp.float32)


def _post_call(aggU, den3, drug_targets, ctlp, dfp2, cid2, fp_table,
               cell_table, W_fp, W_out):
    return pl.pallas_call(
        _post_body,
        out_shape=jax.ShapeDtypeStruct((B, N_NODES), jnp.float32),
    )(aggU, den3, drug_targets, ctlp, dfp2, cid2, fp_table, cell_table,
      W_fp, W_out)


def kernel(ctl, drug_targets, cell_idx, drug_fp, edge_index, edge_weight,
           fp_table, node_emb, W_g, a_s, a_d, W_ctl, W_fp, cell_table, W_out):
    src = edge_index[0]
    dst = edge_index[1]
    a2 = jnp.stack([a_s, a_d], axis=1)
    h, hsd, ctlp = _pre_call(node_emb, W_g, a2, ctl, W_ctl)
    hs = hsd[:, 0]
    hd = hsd[:, 1]
    aggU, den_flat = _edge_call(src, dst, edge_weight, hs, hd, h)
    den3 = den_flat.reshape(NC, N_NODES, 1)
    dfp2 = drug_fp.astype(jnp.int32).reshape(B, 1)
    cid2 = cell_idx.astype(jnp.int32).reshape(B, 1)
    return _post_call(aggU, den3, drug_targets, ctlp, dfp2, cid2, fp_table,
                      cell_table, W_fp, W_out)
